# baseline (speedup 1.0000x reference)
import functools

import jax
import jax.numpy as jnp
import numpy as np
from jax.experimental import pallas as pl
from jax.experimental.pallas import tpu as pltpu
from jax.experimental.shard_map import shard_map
from jax.sharding import Mesh, PartitionSpec as P


def _gate_body(x_ref, w1_hbm, b1_ref, w2_ref, b2_ref, o_ref,
               w1_vmem, stage, sem, *, n_experts, bh, bstage):
    i = pl.program_id(0)
    k = x_ref.shape[1]

    @pl.when(i == 0)
    def _load_w1():
        nslots = stage.shape[0]
        ncopies = w1_hbm.shape[0] // bstage
        for s in range(min(nslots, ncopies)):
            pltpu.make_async_copy(
                w1_hbm.at[s * bstage:(s + 1) * bstage, :],
                stage.at[s], sem.at[s]).start()
        for c in range(ncopies):
            slot = c % nslots
            pltpu.make_async_copy(
                w1_hbm.at[c * bstage:(c + 1) * bstage, :],
                stage.at[slot], sem.at[slot]).wait()
            w1_vmem[c * bstage:(c + 1) * bstage, :] = (
                stage[slot].astype(jnp.bfloat16))
            nxt = c + nslots
            if nxt < ncopies:
                pltpu.make_async_copy(
                    w1_hbm.at[nxt * bstage:(nxt + 1) * bstage, :],
                    stage.at[slot], sem.at[slot]).start()

    xb = x_ref[...].astype(jnp.bfloat16)
    acc = b2_ref[...].astype(jnp.float32)
    for c in range(0, k, bh):
        pre = jax.lax.dot_general(
            xb, w1_vmem[:, c:c + bh], (((1,), (0,)), ((), ())),
            preferred_element_type=jnp.float32,
        ) + b1_ref[:, c:c + bh]
        hc = jnp.tanh(pre).astype(jnp.bfloat16)
        acc = acc + jax.lax.dot_general(
            hc, w2_ref[c:c + bh, :], (((1,), (0,)), ((), ())),
            preferred_element_type=jnp.float32,
        )
    mx = jnp.max(acc, axis=1, keepdims=True)
    iota = jax.lax.broadcasted_iota(jnp.int32, acc.shape, 1)
    idx = jnp.min(jnp.where(acc == mx, iota, n_experts), axis=1,
                  keepdims=True)
    o_ref[...] = (iota == idx).astype(jnp.float32)


def _gate_one_core(x, W1, b1r, W2, b2r):
    m, k = x.shape
    _, h = W1.shape
    e = W2.shape[1]
    bm = min(512, m)
    bh = min(512, h)
    bstage = min(128, h)
    return pl.pallas_call(
        functools.partial(_gate_body, n_experts=e, bh=bh, bstage=bstage),
        grid=(m // bm,),
        in_specs=[
            pl.BlockSpec((bm, k), lambda i: (i, 0)),
            pl.BlockSpec(memory_space=pltpu.MemorySpace.HBM),
            pl.BlockSpec((1, h), lambda i: (0, 0)),
            pl.BlockSpec((h, e), lambda i: (0, 0)),
            pl.BlockSpec((1, e), lambda i: (0, 0)),
        ],
        out_specs=pl.BlockSpec((bm, e), lambda i: (i, 0)),
        out_shape=jax.ShapeDtypeStruct((m, e), jnp.float32),
        scratch_shapes=[
            pltpu.VMEM((k, h), jnp.bfloat16),
            pltpu.VMEM((2, bstage, h), jnp.float32),
            pltpu.SemaphoreType.DMA((2,)),
        ],
        compiler_params=pltpu.CompilerParams(
            dimension_semantics=("arbitrary",),
        ),
    )(x, W1, b1r, W2, b2r)


def kernel(x, W1, b1, W2, b2):
    m = x.shape[0]
    h = W1.shape[1]
    e = W2.shape[1]
    W2 = W2.astype(jnp.bfloat16)
    b1r = b1.reshape(1, h)
    b2r = b2.reshape(1, e)
    devs = jax.devices()
    ndev = len(devs)
    while ndev > 1 and (m % (ndev * 512) != 0):
        ndev -= 1
    if ndev > 1:
        mesh = Mesh(np.array(devs[:ndev]), ("d",))
        sharded = shard_map(
            _gate_one_core, mesh=mesh,
            in_specs=(P("d", None), P(None, None), P(None, None),
                      P(None, None), P(None, None)),
            out_specs=P("d", None),
            check_rep=False,
        )
        return sharded(x, W1, b1r, W2, b2r)
    return _gate_one_core(x, W1, b1r, W2, b2r)


# speedup vs baseline: 1.2672x; 1.2672x over previous
import functools

import jax
import jax.numpy as jnp
from jax.experimental import pallas as pl
from jax.experimental.pallas import tpu as pltpu


def _gate_body(x_ref, w1_hbm, b1_ref, w2_ref, b2_ref, o_ref,
               w1_vmem, stage, sem, acc_sc, *, n_experts, bh, bstage):
    i = pl.program_id(0)
    k = x_ref.shape[1]

    @pl.when(i == 0)
    def _load_w1():
        nslots = stage.shape[0]
        ncopies = w1_hbm.shape[0] // bstage
        for s in range(min(nslots, ncopies)):
            pltpu.make_async_copy(
                w1_hbm.at[s * bstage:(s + 1) * bstage, :],
                stage.at[s], sem.at[s]).start()
        for c in range(ncopies):
            slot = c % nslots
            pltpu.make_async_copy(
                w1_hbm.at[c * bstage:(c + 1) * bstage, :],
                stage.at[slot], sem.at[slot]).wait()
            w1_vmem[c * bstage:(c + 1) * bstage, :] = (
                stage[slot].astype(jnp.bfloat16))
            nxt = c + nslots
            if nxt < ncopies:
                pltpu.make_async_copy(
                    w1_hbm.at[nxt * bstage:(nxt + 1) * bstage, :],
                    stage.at[slot], sem.at[slot]).start()

    last = pl.num_programs(0) - 1

    @pl.when(i < last)
    def _compute():
        xb = x_ref[...].astype(jnp.bfloat16)
        acc = b2_ref[...].astype(jnp.float32)
        for c in range(0, k, bh):
            pre = jax.lax.dot_general(
                xb, w1_vmem[:, c:c + bh], (((1,), (0,)), ((), ())),
                preferred_element_type=jnp.float32,
            ) + b1_ref[:, c:c + bh]
            hc = jnp.tanh(pre).astype(jnp.bfloat16)
            acc = acc + jax.lax.dot_general(
                hc, w2_ref[c:c + bh, :], (((1,), (0,)), ((), ())),
                preferred_element_type=jnp.float32,
            )
        acc_sc[i % 2] = acc

    @pl.when(i > 0)
    def _finalize():
        acc = acc_sc[(i - 1) % 2]
        mx = jnp.max(acc, axis=1, keepdims=True)
        iota = jax.lax.broadcasted_iota(jnp.int32, acc.shape, 1)
        idx = jnp.min(jnp.where(acc == mx, iota, n_experts), axis=1,
                      keepdims=True)
        o_ref[...] = (iota == idx).astype(jnp.float32)


def kernel(x, W1, b1, W2, b2):
    m, k = x.shape
    _, h = W1.shape
    e = W2.shape[1]
    bm = min(512, m)
    bh = min(512, h)
    bstage = min(128, h)
    W2 = W2.astype(jnp.bfloat16)
    b1r = b1.reshape(1, h)
    b2r = b2.reshape(1, e)
    return pl.pallas_call(
        functools.partial(_gate_body, n_experts=e, bh=bh, bstage=bstage),
        grid=(m // bm + 1,),
        in_specs=[
            pl.BlockSpec((bm, k), lambda i: (jnp.minimum(i, m // bm - 1), 0)),
            pl.BlockSpec(memory_space=pltpu.MemorySpace.HBM),
            pl.BlockSpec((1, h), lambda i: (0, 0)),
            pl.BlockSpec((h, e), lambda i: (0, 0)),
            pl.BlockSpec((1, e), lambda i: (0, 0)),
        ],
        out_specs=pl.BlockSpec((bm, e),
                               lambda i: (jnp.maximum(i - 1, 0), 0)),
        out_shape=jax.ShapeDtypeStruct((m, e), jnp.float32),
        scratch_shapes=[
            pltpu.VMEM((k, h), jnp.bfloat16),
            pltpu.VMEM((2, bstage, h), jnp.float32),
            pltpu.SemaphoreType.DMA((2,)),
            pltpu.VMEM((2, bm, e), jnp.float32),
        ],
        compiler_params=pltpu.CompilerParams(
            dimension_semantics=("arbitrary",),
        ),
    )(x, W1, b1r, W2, b2r)


# speedup vs baseline: 1.2719x; 1.0037x over previous
import functools

import jax
import jax.numpy as jnp
from jax.experimental import pallas as pl
from jax.experimental.pallas import tpu as pltpu


def _gate_body(x_ref, w1_hbm, b1_ref, w2_ref, b2_ref, o_ref,
               w1_vmem, stage, sem, *, n_experts, bh, bstage):
    i = pl.program_id(0)
    k = x_ref.shape[1]

    @pl.when(i == 0)
    def _load_w1():
        nslots = stage.shape[0]
        ncopies = w1_hbm.shape[0] // bstage
        for s in range(min(nslots, ncopies)):
            pltpu.make_async_copy(
                w1_hbm.at[s * bstage:(s + 1) * bstage, :],
                stage.at[s], sem.at[s]).start()
        for c in range(ncopies):
            slot = c % nslots
            pltpu.make_async_copy(
                w1_hbm.at[c * bstage:(c + 1) * bstage, :],
                stage.at[slot], sem.at[slot]).wait()
            w1_vmem[c * bstage:(c + 1) * bstage, :] = (
                stage[slot].astype(jnp.bfloat16))
            nxt = c + nslots
            if nxt < ncopies:
                pltpu.make_async_copy(
                    w1_hbm.at[nxt * bstage:(nxt + 1) * bstage, :],
                    stage.at[slot], sem.at[slot]).start()

    xb = x_ref[...].astype(jnp.bfloat16)
    acc = b2_ref[...].astype(jnp.float32)
    for c in range(0, k, bh):
        pre = jax.lax.dot_general(
            xb, w1_vmem[:, c:c + bh], (((1,), (0,)), ((), ())),
            preferred_element_type=jnp.float32,
        ) + b1_ref[:, c:c + bh]
        hc = jnp.tanh(pre).astype(jnp.bfloat16)
        acc = acc + jax.lax.dot_general(
            hc, w2_ref[c:c + bh, :], (((1,), (0,)), ((), ())),
            preferred_element_type=jnp.float32,
        )
    mx = jnp.max(acc, axis=1, keepdims=True)
    iota = jax.lax.broadcasted_iota(jnp.int32, acc.shape, 1)
    idx = jnp.min(jnp.where(acc == mx, iota, n_experts), axis=1,
                  keepdims=True)
    o_ref[...] = (iota == idx).astype(jnp.float32)


def kernel(x, W1, b1, W2, b2):
    m, k = x.shape
    _, h = W1.shape
    e = W2.shape[1]
    bm = min(512, m)
    bh = min(512, h)
    bstage = min(128, h)
    W2 = W2.astype(jnp.bfloat16)
    b1r = b1.reshape(1, h)
    b2r = b2.reshape(1, e)
    return pl.pallas_call(
        functools.partial(_gate_body, n_experts=e, bh=bh, bstage=bstage),
        grid=(m // bm,),
        in_specs=[
            pl.BlockSpec((bm, k), lambda i: (i, 0)),
            pl.BlockSpec(memory_space=pltpu.MemorySpace.HBM),
            pl.BlockSpec((1, h), lambda i: (0, 0)),
            pl.BlockSpec((h, e), lambda i: (0, 0)),
            pl.BlockSpec((1, e), lambda i: (0, 0)),
        ],
        out_specs=pl.BlockSpec((bm, e), lambda i: (i, 0)),
        out_shape=jax.ShapeDtypeStruct((m, e), jnp.float32),
        scratch_shapes=[
            pltpu.VMEM((k, h), jnp.bfloat16),
            pltpu.VMEM((2, bstage, h), jnp.float32),
            pltpu.SemaphoreType.DMA((2,)),
        ],
        compiler_params=pltpu.CompilerParams(
            dimension_semantics=("arbitrary",),
        ),
    )(x, W1, b1r, W2, b2r)
